# baseline (speedup 1.0000x reference)
import jax
import jax.numpy as jnp
from jax import lax
from jax.experimental import pallas as pl
from jax.experimental.pallas import tpu as pltpu
from jax.experimental.pallas import tpu_sc as plsc

N = 10000
E = 320000
D_IN = 128
D_HID = 64

NC, NS = 2, 16
NW = NC * NS
EPW = E // NW
CH = 80
NCHUNK = 128
EPWP = NCHUNK * CH
PADW = EPWP - EPW
NB = 4
ROUNDS = NCHUNK // NB
NP = 10240
RP = NP // NS

_mesh = plsc.VectorSubcoreMesh(core_axis_name="c", subcore_axis_name="s")


def _proj1_body(x_ref, wl_ref, wr_ref, y1_ref, r1_ref):
    x = x_ref[...]
    dn = (((1,), (1,)), ((), ()))
    y1_ref[...] = lax.dot_general(x, wl_ref[...], dn,
                                  preferred_element_type=jnp.float32)
    r1_ref[...] = lax.dot_general(x, wr_ref[...], dn,
                                  preferred_element_type=jnp.float32)


def _agg1_body(y1_hbm, src_hbm, dst_hbm, z64_hbm, z1_hbm, one_hbm,
               p_hbm, cnt_hbm,
               agg_sh, cnt_sh, src_v, dst_v, rows_v, ones_v, stage_v,
               cstage_v, g0, g1, s0, s1, csem):
    c = lax.axis_index("c")
    s = lax.axis_index("s")
    wid = c * NS + s

    pltpu.sync_copy(z64_hbm, stage_v)
    pltpu.sync_copy(z1_hbm, cstage_v)
    pltpu.sync_copy(stage_v, agg_sh.at[pl.ds(s * RP, RP)])
    pltpu.sync_copy(cstage_v, cnt_sh.at[pl.ds(s * RP, RP)])
    pltpu.sync_copy(one_hbm, ones_v)
    pltpu.sync_copy(src_hbm.at[wid], src_v)
    pltpu.sync_copy(dst_hbm.at[wid], dst_v)
    pltpu.async_copy(y1_hbm.at[src_v.at[0]], rows_v.at[0], g0)
    plsc.subcore_barrier()

    def step(i, carry):
        a = 2 * i + 1
        pltpu.async_copy(y1_hbm.at[src_v.at[a]], rows_v.at[1], g1)
        pltpu.make_async_copy(y1_hbm.at[src_v.at[0]], rows_v.at[0], g0).wait()
        pltpu.async_copy(rows_v.at[0], agg_sh.at[dst_v.at[a - 1]], s0,
                         add=True)
        pltpu.async_copy(ones_v, cnt_sh.at[dst_v.at[a - 1]], csem, add=True)
        pltpu.make_async_copy(rows_v.at[0], agg_sh.at[dst_v.at[0]], s0).wait()
        nxt = jnp.where(a + 1 < NCHUNK, a + 1, 0)
        pltpu.async_copy(y1_hbm.at[src_v.at[nxt]], rows_v.at[0], g0)
        pltpu.make_async_copy(y1_hbm.at[src_v.at[0]], rows_v.at[1], g1).wait()
        pltpu.async_copy(rows_v.at[1], agg_sh.at[dst_v.at[a]], s1, add=True)
        pltpu.async_copy(ones_v, cnt_sh.at[dst_v.at[a]], csem, add=True)
        pltpu.make_async_copy(rows_v.at[1], agg_sh.at[dst_v.at[0]], s1).wait()
        pltpu.make_async_copy(ones_v, cnt_sh.at[dst_v.at[0]], csem).wait()
        pltpu.make_async_copy(ones_v, cnt_sh.at[dst_v.at[0]], csem).wait()
        return carry

    lax.fori_loop(0, NCHUNK // 2, step, 0)
    pltpu.make_async_copy(y1_hbm.at[src_v.at[0]], rows_v.at[0], g0).wait()
    plsc.subcore_barrier()

    pltpu.sync_copy(agg_sh.at[pl.ds(s * RP, RP)], stage_v)
    pltpu.sync_copy(cnt_sh.at[pl.ds(s * RP, RP)], cstage_v)
    pltpu.sync_copy(stage_v, p_hbm.at[c, pl.ds(s * RP, RP)])
    pltpu.sync_copy(cstage_v, cnt_hbm.at[c, pl.ds(s * RP, RP)])


def _mid_body(p_ref, cnt_ref, r1_ref, b1_ref, w2l_ref, w2r_ref,
              y2_ref, r2_ref):
    cnt = cnt_ref[0, :N] + cnt_ref[1, :N]
    rcp = 1.0 / jnp.maximum(cnt, 1.0)
    agg = p_ref[0, :N] + p_ref[1, :N]
    h = jax.nn.relu(agg * rcp[:, None] + r1_ref[...] + b1_ref[...][None, :])
    pad = jnp.zeros((NP - N,), jnp.float32)
    y2_ref[...] = jnp.concatenate(
        [jnp.sum(h * w2l_ref[...][0][None, :], axis=1), pad])
    r2_ref[...] = jnp.sum(h * w2r_ref[...][0][None, :], axis=1)


def _agg2_body(y2_hbm, src_hbm, dst_hbm, z1_hbm,
               q_hbm,
               q_sh, src_v, dst_v, vals_v, y2l_v, cstage_v,
               s0, s1, s2, s3):
    ssem = (s0, s1, s2, s3)
    c = lax.axis_index("c")
    s = lax.axis_index("s")
    wid = c * NS + s

    pltpu.sync_copy(z1_hbm, cstage_v)
    pltpu.sync_copy(cstage_v, q_sh.at[pl.ds(s * RP, RP)])
    pltpu.sync_copy(y2_hbm, y2l_v)
    pltpu.sync_copy(src_hbm.at[wid], src_v)
    pltpu.sync_copy(dst_hbm.at[wid], dst_v)
    plsc.subcore_barrier()

    def fill_and_scatter(ch, b):
        sv = src_v.at[ch]
        vb = vals_v.at[b]
        for j in range(CH // 16):
            idx = sv[pl.ds(j * 16, 16)]
            vb[pl.ds(j * 16, 16)] = plsc.load_gather(y2l_v, [idx])
        pltpu.async_copy(vb, q_sh.at[dst_v.at[ch]], ssem[b], add=True)

    for b in range(NB):
        fill_and_scatter(b, b)

    def round_(i, carry):
        for b in range(NB):
            ch = i * NB + b
            pltpu.make_async_copy(vals_v.at[b], q_sh.at[dst_v.at[0]],
                                  ssem[b]).wait()
            fill_and_scatter(ch, b)
        return carry

    lax.fori_loop(1, ROUNDS, round_, 0)
    for b in range(NB):
        pltpu.make_async_copy(vals_v.at[b], q_sh.at[dst_v.at[0]],
                              ssem[b]).wait()
    plsc.subcore_barrier()

    pltpu.sync_copy(q_sh.at[pl.ds(s * RP, RP)], cstage_v)
    pltpu.sync_copy(cstage_v, q_hbm.at[c, pl.ds(s * RP, RP)])


def _out_body(q_ref, cnt_ref, r2_ref, b2_ref, out_ref):
    cnt = cnt_ref[0, :N] + cnt_ref[1, :N]
    rcp = 1.0 / jnp.maximum(cnt, 1.0)
    z = (q_ref[0, :N] + q_ref[1, :N]) * rcp + r2_ref[...] + b2_ref[0]
    out_ref[...] = jax.nn.sigmoid(z)[:, None]


@jax.jit
def kernel(x, edge_index, W1l, W1r, b1, W2l, W2r, b2):
    f32 = jnp.float32
    i32 = jnp.int32
    src = jnp.concatenate(
        [edge_index[0].reshape(NW, EPW), jnp.zeros((NW, PADW), i32)],
        axis=1).reshape(NW, NCHUNK, CH)
    dst = jnp.concatenate(
        [edge_index[1].reshape(NW, EPW), jnp.full((NW, PADW), N, i32)],
        axis=1).reshape(NW, NCHUNK, CH)
    z64 = jnp.zeros((RP, D_HID), f32)
    z1 = jnp.zeros((RP,), f32)
    ones = jnp.ones((CH,), f32)

    y1, r1 = pl.pallas_call(
        _proj1_body,
        out_shape=[jax.ShapeDtypeStruct((N, D_HID), f32),
                   jax.ShapeDtypeStruct((N, D_HID), f32)],
    )(x, W1l, W1r)

    agg1_partial, cnt_partial = pl.kernel(
        _agg1_body,
        out_type=[jax.ShapeDtypeStruct((NC, NP, D_HID), f32),
                  jax.ShapeDtypeStruct((NC, NP), f32)],
        mesh=_mesh,
        compiler_params=pltpu.CompilerParams(use_tc_tiling_on_sc=False),
        scratch_types=[
            pltpu.VMEM_SHARED((NP, D_HID), f32),
            pltpu.VMEM_SHARED((NP,), f32),
            pltpu.VMEM((NCHUNK, CH), i32),
            pltpu.VMEM((NCHUNK, CH), i32),
            pltpu.VMEM((2, CH, D_HID), f32),
            pltpu.VMEM((CH,), f32),
            pltpu.VMEM((RP, D_HID), f32),
            pltpu.VMEM((RP,), f32),
        ] + [pltpu.SemaphoreType.DMA] * 5,
    )(y1, src, dst, z64, z1, ones)

    y2, r2 = pl.pallas_call(
        _mid_body,
        out_shape=[jax.ShapeDtypeStruct((NP,), f32),
                   jax.ShapeDtypeStruct((N,), f32)],
    )(agg1_partial, cnt_partial, r1, b1, W2l, W2r)

    q_partial = pl.kernel(
        _agg2_body,
        out_type=jax.ShapeDtypeStruct((NC, NP), f32),
        mesh=_mesh,
        compiler_params=pltpu.CompilerParams(use_tc_tiling_on_sc=False,
                                             needs_layout_passes=False),
        scratch_types=[
            pltpu.VMEM_SHARED((NP,), f32),
            pltpu.VMEM((NCHUNK, CH), i32),
            pltpu.VMEM((NCHUNK, CH), i32),
            pltpu.VMEM((NB, CH), f32),
            pltpu.VMEM((NP,), f32),
            pltpu.VMEM((RP,), f32),
        ] + [pltpu.SemaphoreType.DMA] * 4,
    )(y2, src, dst, z1)

    out = pl.pallas_call(
        _out_body,
        out_shape=jax.ShapeDtypeStruct((N, 1), f32),
    )(q_partial, cnt_partial, r2, b2)

    return out


# speedup vs baseline: 1.0005x; 1.0005x over previous
import jax
import jax.numpy as jnp
from jax import lax
from jax.experimental import pallas as pl
from jax.experimental.pallas import tpu as pltpu
from jax.experimental.pallas import tpu_sc as plsc

N = 10000
E = 320000
D_IN = 128
D_HID = 64

NC, NS = 2, 16
NW = NC * NS
EPW = E // NW
CH = 80
NCHUNK = 128
EPWP = NCHUNK * CH
PADW = EPWP - EPW
NB = 4
ROUNDS = NCHUNK // NB
NP = 10240
RP = NP // NS

_mesh = plsc.VectorSubcoreMesh(core_axis_name="c", subcore_axis_name="s")


def _proj1_body(x_ref, wl_ref, wr_ref, y1_ref, r1_ref):
    x = x_ref[...]
    dn = (((1,), (1,)), ((), ()))
    y1_ref[...] = lax.dot_general(x, wl_ref[...], dn,
                                  preferred_element_type=jnp.float32)
    r1_ref[...] = lax.dot_general(x, wr_ref[...], dn,
                                  preferred_element_type=jnp.float32)


def _agg1_body(y1_hbm, src_hbm, dst_hbm, z64_hbm, z1_hbm, one_hbm,
               p_hbm, cnt_hbm,
               agg_sh, cnt_sh, src_v, dst_v, rows0_v, rows1_v, ones_v,
               stage_v, cstage_v, g0, g1, s0, s1, csem):
    c = lax.axis_index("c")
    s = lax.axis_index("s")
    wid = c * NS + s

    pltpu.sync_copy(z64_hbm, stage_v)
    pltpu.sync_copy(z1_hbm, cstage_v)
    pltpu.sync_copy(stage_v, agg_sh.at[pl.ds(s * RP, RP)])
    pltpu.sync_copy(cstage_v, cnt_sh.at[pl.ds(s * RP, RP)])
    pltpu.sync_copy(one_hbm, ones_v)
    pltpu.sync_copy(src_hbm.at[wid], src_v)
    pltpu.sync_copy(dst_hbm.at[wid], dst_v)
    pltpu.async_copy(y1_hbm.at[src_v.at[0]], rows0_v, g0)
    plsc.subcore_barrier()

    def step(i, carry):
        a = 2 * i + 1
        pltpu.async_copy(y1_hbm.at[src_v.at[a]], rows1_v, g1)
        pltpu.make_async_copy(y1_hbm.at[src_v.at[0]], rows0_v, g0).wait()
        pltpu.async_copy(rows0_v, agg_sh.at[dst_v.at[a - 1]], s0,
                         add=True)
        pltpu.async_copy(ones_v, cnt_sh.at[dst_v.at[a - 1]], csem, add=True)
        pltpu.make_async_copy(rows0_v, agg_sh.at[dst_v.at[0]], s0).wait()
        nxt = jnp.where(a + 1 < NCHUNK, a + 1, 0)
        pltpu.async_copy(y1_hbm.at[src_v.at[nxt]], rows0_v, g0)
        pltpu.make_async_copy(y1_hbm.at[src_v.at[0]], rows1_v, g1).wait()
        pltpu.async_copy(rows1_v, agg_sh.at[dst_v.at[a]], s1, add=True)
        pltpu.async_copy(ones_v, cnt_sh.at[dst_v.at[a]], csem, add=True)
        pltpu.make_async_copy(rows1_v, agg_sh.at[dst_v.at[0]], s1).wait()
        pltpu.make_async_copy(ones_v, cnt_sh.at[dst_v.at[0]], csem).wait()
        pltpu.make_async_copy(ones_v, cnt_sh.at[dst_v.at[0]], csem).wait()
        return carry

    lax.fori_loop(0, NCHUNK // 2, step, 0)
    pltpu.make_async_copy(y1_hbm.at[src_v.at[0]], rows0_v, g0).wait()
    plsc.subcore_barrier()

    pltpu.sync_copy(agg_sh.at[pl.ds(s * RP, RP)], stage_v)
    pltpu.sync_copy(cnt_sh.at[pl.ds(s * RP, RP)], cstage_v)
    pltpu.sync_copy(stage_v, p_hbm.at[c, pl.ds(s * RP, RP)])
    pltpu.sync_copy(cstage_v, cnt_hbm.at[c, pl.ds(s * RP, RP)])


def _mid_body(p_ref, cnt_ref, r1_ref, b1_ref, w2l_ref, w2r_ref,
              y2_ref, r2_ref):
    cnt = cnt_ref[0, :N] + cnt_ref[1, :N]
    rcp = 1.0 / jnp.maximum(cnt, 1.0)
    agg = p_ref[0, :N] + p_ref[1, :N]
    h = jax.nn.relu(agg * rcp[:, None] + r1_ref[...] + b1_ref[...][None, :])
    pad = jnp.zeros((NP - N,), jnp.float32)
    y2_ref[...] = jnp.concatenate(
        [jnp.sum(h * w2l_ref[...][0][None, :], axis=1), pad])
    r2_ref[...] = jnp.sum(h * w2r_ref[...][0][None, :], axis=1)


def _agg2_body(y2_hbm, src_hbm, dst_hbm, z1_hbm,
               q_hbm,
               q_sh, src_v, dst_v, vals_v, y2l_v, cstage_v,
               s0, s1, s2, s3):
    ssem = (s0, s1, s2, s3)
    c = lax.axis_index("c")
    s = lax.axis_index("s")
    wid = c * NS + s

    pltpu.sync_copy(z1_hbm, cstage_v)
    pltpu.sync_copy(cstage_v, q_sh.at[pl.ds(s * RP, RP)])
    pltpu.sync_copy(y2_hbm, y2l_v)
    pltpu.sync_copy(src_hbm.at[wid], src_v)
    pltpu.sync_copy(dst_hbm.at[wid], dst_v)
    plsc.subcore_barrier()

    def fill_and_scatter(ch, b):
        sv = src_v.at[ch]
        vb = vals_v.at[b]
        for j in range(CH // 16):
            idx = sv[pl.ds(j * 16, 16)]
            vb[pl.ds(j * 16, 16)] = plsc.load_gather(y2l_v, [idx])
        pltpu.async_copy(vb, q_sh.at[dst_v.at[ch]], ssem[b], add=True)

    for b in range(NB):
        fill_and_scatter(b, b)

    def round_(i, carry):
        for b in range(NB):
            ch = i * NB + b
            pltpu.make_async_copy(vals_v.at[b], q_sh.at[dst_v.at[0]],
                                  ssem[b]).wait()
            fill_and_scatter(ch, b)
        return carry

    lax.fori_loop(1, ROUNDS, round_, 0)
    for b in range(NB):
        pltpu.make_async_copy(vals_v.at[b], q_sh.at[dst_v.at[0]],
                              ssem[b]).wait()
    plsc.subcore_barrier()

    pltpu.sync_copy(q_sh.at[pl.ds(s * RP, RP)], cstage_v)
    pltpu.sync_copy(cstage_v, q_hbm.at[c, pl.ds(s * RP, RP)])


def _out_body(q_ref, cnt_ref, r2_ref, b2_ref, out_ref):
    cnt = cnt_ref[0, :N] + cnt_ref[1, :N]
    rcp = 1.0 / jnp.maximum(cnt, 1.0)
    z = (q_ref[0, :N] + q_ref[1, :N]) * rcp + r2_ref[...] + b2_ref[0]
    out_ref[...] = jax.nn.sigmoid(z)[:, None]


@jax.jit
def kernel(x, edge_index, W1l, W1r, b1, W2l, W2r, b2):
    f32 = jnp.float32
    i32 = jnp.int32
    src = jnp.concatenate(
        [edge_index[0].reshape(NW, EPW), jnp.zeros((NW, PADW), i32)],
        axis=1).reshape(NW, NCHUNK, CH)
    dst = jnp.concatenate(
        [edge_index[1].reshape(NW, EPW), jnp.full((NW, PADW), N, i32)],
        axis=1).reshape(NW, NCHUNK, CH)
    z64 = jnp.zeros((RP, D_HID), f32)
    z1 = jnp.zeros((RP,), f32)
    ones = jnp.ones((CH,), f32)

    y1, r1 = pl.pallas_call(
        _proj1_body,
        out_shape=[jax.ShapeDtypeStruct((N, D_HID), f32),
                   jax.ShapeDtypeStruct((N, D_HID), f32)],
    )(x, W1l, W1r)

    agg1_partial, cnt_partial = pl.kernel(
        _agg1_body,
        out_type=[jax.ShapeDtypeStruct((NC, NP, D_HID), f32),
                  jax.ShapeDtypeStruct((NC, NP), f32)],
        mesh=_mesh,
        compiler_params=pltpu.CompilerParams(use_tc_tiling_on_sc=False),
        scratch_types=[
            pltpu.VMEM_SHARED((NP, D_HID), f32),
            pltpu.VMEM_SHARED((NP,), f32),
            pltpu.VMEM((NCHUNK, CH), i32),
            pltpu.VMEM((NCHUNK, CH), i32),
            pltpu.VMEM((CH, D_HID), f32),
            pltpu.VMEM((CH, D_HID), f32),
            pltpu.VMEM((CH,), f32),
            pltpu.VMEM((RP, D_HID), f32),
            pltpu.VMEM((RP,), f32),
        ] + [pltpu.SemaphoreType.DMA] * 5,
    )(y1, src, dst, z64, z1, ones)

    y2, r2 = pl.pallas_call(
        _mid_body,
        out_shape=[jax.ShapeDtypeStruct((NP,), f32),
                   jax.ShapeDtypeStruct((N,), f32)],
    )(agg1_partial, cnt_partial, r1, b1, W2l, W2r)

    q_partial = pl.kernel(
        _agg2_body,
        out_type=jax.ShapeDtypeStruct((NC, NP), f32),
        mesh=_mesh,
        compiler_params=pltpu.CompilerParams(use_tc_tiling_on_sc=False,
                                             needs_layout_passes=False),
        scratch_types=[
            pltpu.VMEM_SHARED((NP,), f32),
            pltpu.VMEM((NCHUNK, CH), i32),
            pltpu.VMEM((NCHUNK, CH), i32),
            pltpu.VMEM((NB, CH), f32),
            pltpu.VMEM((NP,), f32),
            pltpu.VMEM((RP,), f32),
        ] + [pltpu.SemaphoreType.DMA] * 4,
    )(y2, src, dst, z1)

    out = pl.pallas_call(
        _out_body,
        out_shape=jax.ShapeDtypeStruct((N, 1), f32),
    )(q_partial, cnt_partial, r2, b2)

    return out


# speedup vs baseline: 1.0125x; 1.0119x over previous
import jax
import jax.numpy as jnp
from jax import lax
from jax.experimental import pallas as pl
from jax.experimental.pallas import tpu as pltpu
from jax.experimental.pallas import tpu_sc as plsc

N = 10000
E = 320000
D_IN = 128
D_HID = 64

NC, NS = 2, 16
NW = NC * NS
EPW = E // NW
CH = 80
NCHUNK = 128
EPWP = NCHUNK * CH
PADW = EPWP - EPW
NB = 4
ROUNDS = NCHUNK // NB
NP = 10240
RP = NP // NS

_mesh = plsc.VectorSubcoreMesh(core_axis_name="c", subcore_axis_name="s")


def _proj1_body(x_ref, wl_ref, wr_ref, y1_ref, r1_ref):
    x = x_ref[...]
    dn = (((1,), (1,)), ((), ()))
    y1_ref[...] = lax.dot_general(x, wl_ref[...], dn,
                                  preferred_element_type=jnp.float32)
    r1_ref[...] = lax.dot_general(x, wr_ref[...], dn,
                                  preferred_element_type=jnp.float32)


def _agg1_body(y1_hbm, src_hbm, dst_hbm, z64_hbm, z1_hbm, one_hbm,
               p_hbm, cnt_hbm,
               agg_sh, cnt_sh, src_v, dst_v, rows0_v, rows1_v, ones_v,
               stage_v, cstage_v, g0, g1, s0, s1, csem):
    c = lax.axis_index("c")
    s = lax.axis_index("s")
    wid = c * NS + s

    pltpu.sync_copy(z64_hbm, stage_v)
    pltpu.sync_copy(z1_hbm, cstage_v)
    pltpu.sync_copy(stage_v, agg_sh.at[pl.ds(s * RP, RP)])
    pltpu.sync_copy(cstage_v, cnt_sh.at[pl.ds(s * RP, RP)])
    pltpu.sync_copy(one_hbm, ones_v)
    pltpu.sync_copy(src_hbm.at[wid], src_v)
    pltpu.sync_copy(dst_hbm.at[wid], dst_v)
    pltpu.async_copy(y1_hbm.at[src_v.at[0]], rows0_v, g0)
    plsc.subcore_barrier()

    def step(i, carry):
        a = 2 * i + 1
        pltpu.async_copy(y1_hbm.at[src_v.at[a]], rows1_v, g1)
        pltpu.make_async_copy(y1_hbm.at[src_v.at[0]], rows0_v, g0).wait()
        pltpu.async_copy(rows0_v, agg_sh.at[dst_v.at[a - 1]], s0,
                         add=True)
        pltpu.async_copy(ones_v, cnt_sh.at[dst_v.at[a - 1]], csem, add=True)
        pltpu.make_async_copy(rows0_v, agg_sh.at[dst_v.at[0]], s0).wait()
        nxt = jnp.where(a + 1 < NCHUNK, a + 1, 0)
        pltpu.async_copy(y1_hbm.at[src_v.at[nxt]], rows0_v, g0)
        pltpu.make_async_copy(y1_hbm.at[src_v.at[0]], rows1_v, g1).wait()
        pltpu.async_copy(rows1_v, agg_sh.at[dst_v.at[a]], s1, add=True)
        pltpu.async_copy(ones_v, cnt_sh.at[dst_v.at[a]], csem, add=True)
        pltpu.make_async_copy(rows1_v, agg_sh.at[dst_v.at[0]], s1).wait()
        pltpu.make_async_copy(ones_v, cnt_sh.at[dst_v.at[0]], csem).wait()
        pltpu.make_async_copy(ones_v, cnt_sh.at[dst_v.at[0]], csem).wait()
        return carry

    lax.fori_loop(0, NCHUNK // 2, step, 0)
    pltpu.make_async_copy(y1_hbm.at[src_v.at[0]], rows0_v, g0).wait()
    plsc.subcore_barrier()

    pltpu.sync_copy(agg_sh.at[pl.ds(s * RP, RP)], stage_v)
    pltpu.sync_copy(cnt_sh.at[pl.ds(s * RP, RP)], cstage_v)
    pltpu.sync_copy(stage_v, p_hbm.at[c, pl.ds(s * RP, RP)])
    pltpu.sync_copy(cstage_v, cnt_hbm.at[c, pl.ds(s * RP, RP)])


def _mid_body(p_ref, cnt_ref, r1_ref, b1_ref, w2l_ref, w2r_ref,
              y2_ref, r2_ref):
    cnt = cnt_ref[0, :N] + cnt_ref[1, :N]
    rcp = 1.0 / jnp.maximum(cnt, 1.0)
    agg = p_ref[0, :N] + p_ref[1, :N]
    h = jax.nn.relu(agg * rcp[:, None] + r1_ref[...] + b1_ref[...][None, :])
    pad = jnp.zeros((NP - N,), jnp.float32)
    y2_ref[...] = jnp.concatenate(
        [jnp.sum(h * w2l_ref[...][0][None, :], axis=1), pad])
    r2_ref[...] = jnp.sum(h * w2r_ref[...][0][None, :], axis=1)


def _agg2_body(y2_hbm, src_hbm, dst_hbm, z1_hbm,
               q_hbm,
               q_sh, src_v, dst_v, vals_v, y2l_v, cstage_v,
               s0, s1, s2, s3):
    ssem = (s0, s1, s2, s3)
    c = lax.axis_index("c")
    s = lax.axis_index("s")
    wid = c * NS + s

    pltpu.sync_copy(z1_hbm, cstage_v)
    pltpu.sync_copy(cstage_v, q_sh.at[pl.ds(s * RP, RP)])
    pltpu.sync_copy(y2_hbm, y2l_v)
    pltpu.sync_copy(src_hbm.at[wid], src_v)
    pltpu.sync_copy(dst_hbm.at[wid], dst_v)
    plsc.subcore_barrier()

    def fill_and_scatter(ch, b):
        sv = src_v.at[ch]
        vb = vals_v.at[b]
        for j in range(CH // 16):
            idx = sv[pl.ds(j * 16, 16)]
            vb[pl.ds(j * 16, 16)] = plsc.load_gather(y2l_v, [idx])
        pltpu.async_copy(vb, q_sh.at[dst_v.at[ch]], ssem[b], add=True)

    for b in range(NB):
        fill_and_scatter(b, b)

    def round_(i, carry):
        for b in range(NB):
            ch = i * NB + b
            pltpu.make_async_copy(vals_v.at[b], q_sh.at[dst_v.at[0]],
                                  ssem[b]).wait()
            fill_and_scatter(ch, b)
        return carry

    lax.fori_loop(1, ROUNDS, round_, 0)
    for b in range(NB):
        pltpu.make_async_copy(vals_v.at[b], q_sh.at[dst_v.at[0]],
                              ssem[b]).wait()
    plsc.subcore_barrier()

    pltpu.sync_copy(q_sh.at[pl.ds(s * RP, RP)], cstage_v)
    pltpu.sync_copy(cstage_v, q_hbm.at[c, pl.ds(s * RP, RP)])


def _out_body(q_ref, cnt_ref, r2_ref, b2_ref, out_ref):
    cnt = cnt_ref[0, :N] + cnt_ref[1, :N]
    rcp = 1.0 / jnp.maximum(cnt, 1.0)
    z = (q_ref[0, :N] + q_ref[1, :N]) * rcp + r2_ref[...] + b2_ref[0]
    out_ref[...] = jax.nn.sigmoid(z)[:, None]


@jax.jit
def kernel(x, edge_index, W1l, W1r, b1, W2l, W2r, b2):
    f32 = jnp.float32
    i32 = jnp.int32
    src = jnp.concatenate(
        [edge_index[0].reshape(NW, EPW), jnp.zeros((NW, PADW), i32)],
        axis=1).reshape(NW, NCHUNK, CH)
    pad_dst = jnp.broadcast_to(jnp.arange(N, NP, dtype=i32)[None, :],
                               (NW, PADW))
    dst = jnp.concatenate(
        [edge_index[1].reshape(NW, EPW), pad_dst],
        axis=1).reshape(NW, NCHUNK, CH)
    z64 = jnp.zeros((RP, D_HID), f32)
    z1 = jnp.zeros((RP,), f32)
    ones = jnp.ones((CH,), f32)

    y1, r1 = pl.pallas_call(
        _proj1_body,
        out_shape=[jax.ShapeDtypeStruct((N, D_HID), f32),
                   jax.ShapeDtypeStruct((N, D_HID), f32)],
    )(x, W1l, W1r)

    agg1_partial, cnt_partial = pl.kernel(
        _agg1_body,
        out_type=[jax.ShapeDtypeStruct((NC, NP, D_HID), f32),
                  jax.ShapeDtypeStruct((NC, NP), f32)],
        mesh=_mesh,
        compiler_params=pltpu.CompilerParams(use_tc_tiling_on_sc=False),
        scratch_types=[
            pltpu.VMEM_SHARED((NP, D_HID), f32),
            pltpu.VMEM_SHARED((NP,), f32),
            pltpu.VMEM((NCHUNK, CH), i32),
            pltpu.VMEM((NCHUNK, CH), i32),
            pltpu.VMEM((CH, D_HID), f32),
            pltpu.VMEM((CH, D_HID), f32),
            pltpu.VMEM((CH,), f32),
            pltpu.VMEM((RP, D_HID), f32),
            pltpu.VMEM((RP,), f32),
        ] + [pltpu.SemaphoreType.DMA] * 5,
    )(y1, src, dst, z64, z1, ones)

    y2, r2 = pl.pallas_call(
        _mid_body,
        out_shape=[jax.ShapeDtypeStruct((NP,), f32),
                   jax.ShapeDtypeStruct((N,), f32)],
    )(agg1_partial, cnt_partial, r1, b1, W2l, W2r)

    q_partial = pl.kernel(
        _agg2_body,
        out_type=jax.ShapeDtypeStruct((NC, NP), f32),
        mesh=_mesh,
        compiler_params=pltpu.CompilerParams(use_tc_tiling_on_sc=False,
                                             needs_layout_passes=False),
        scratch_types=[
            pltpu.VMEM_SHARED((NP,), f32),
            pltpu.VMEM((NCHUNK, CH), i32),
            pltpu.VMEM((NCHUNK, CH), i32),
            pltpu.VMEM((NB, CH), f32),
            pltpu.VMEM((NP,), f32),
            pltpu.VMEM((RP,), f32),
        ] + [pltpu.SemaphoreType.DMA] * 4,
    )(y2, src, dst, z1)

    out = pl.pallas_call(
        _out_body,
        out_shape=jax.ShapeDtypeStruct((N, 1), f32),
    )(q_partial, cnt_partial, r2, b2)

    return out


# speedup vs baseline: 1.8021x; 1.7799x over previous
import jax
import jax.numpy as jnp
from jax import lax
from jax.experimental import pallas as pl
from jax.experimental.pallas import tpu as pltpu
from jax.experimental.pallas import tpu_sc as plsc

N = 10000
E = 320000
D_IN = 128
D_HID = 64

NC, NS = 2, 16
NW = NC * NS
EPW = E // NW
CH = 80
NCHUNK = EPW // CH
NCHUNKP = 128
EPWP = NCHUNKP * CH
PADW = EPWP - EPW
NB = 4
ROUNDS = NCHUNKP // NB
NP = 10240
RP = NP // NS

_mesh = plsc.VectorSubcoreMesh(core_axis_name="c", subcore_axis_name="s")


def _proj1_body(x_ref, wl_ref, wr_ref, y1_ref, r1_ref):
    x = x_ref[...]
    dn = (((1,), (1,)), ((), ()))
    y1_ref[...] = lax.dot_general(x, wl_ref[...], dn,
                                  preferred_element_type=jnp.float32)
    r1_ref[...] = lax.dot_general(x, wr_ref[...], dn,
                                  preferred_element_type=jnp.float32)


def _agg1_body(y1_hbm, src_hbm, dst_hbm, z64_hbm, z1_hbm, one_hbm,
               p_hbm, cnt_hbm,
               agg_sh, cnt_sh, src_v, dst_v, rows0_v, rows1_v, ones_v,
               stage_v, cstage_v, gsem0, gsem1, ssem0, ssem1, csem):
    c = lax.axis_index("c")
    s = lax.axis_index("s")
    wid = c * NS + s

    pltpu.sync_copy(z64_hbm, stage_v)
    pltpu.sync_copy(z1_hbm, cstage_v)
    pltpu.sync_copy(stage_v, agg_sh.at[pl.ds(s * RP, RP)])
    pltpu.sync_copy(cstage_v, cnt_sh.at[pl.ds(s * RP, RP)])
    pltpu.sync_copy(one_hbm, ones_v)
    pltpu.sync_copy(src_hbm.at[wid], src_v)
    pltpu.sync_copy(dst_hbm.at[wid], dst_v)
    pltpu.async_copy(y1_hbm.at[src_v.at[0]], rows0_v, gsem0)
    plsc.subcore_barrier()

    def step(i, carry):
        a = 2 * i + 1
        pltpu.async_copy(y1_hbm.at[src_v.at[a]], rows1_v, gsem1)
        pltpu.make_async_copy(y1_hbm.at[src_v.at[0]], rows0_v, gsem0).wait()
        pltpu.async_copy(rows0_v, agg_sh.at[dst_v.at[a - 1]], ssem0, add=True)
        pltpu.async_copy(ones_v, cnt_sh.at[dst_v.at[a - 1]], csem, add=True)
        pltpu.make_async_copy(rows0_v, agg_sh.at[dst_v.at[0]], ssem0).wait()
        pltpu.async_copy(y1_hbm.at[src_v.at[a + 1]], rows0_v, gsem0)
        pltpu.make_async_copy(y1_hbm.at[src_v.at[0]], rows1_v, gsem1).wait()
        pltpu.async_copy(rows1_v, agg_sh.at[dst_v.at[a]], ssem1, add=True)
        pltpu.async_copy(ones_v, cnt_sh.at[dst_v.at[a]], csem, add=True)
        pltpu.make_async_copy(rows1_v, agg_sh.at[dst_v.at[0]], ssem1).wait()
        pltpu.make_async_copy(ones_v, cnt_sh.at[dst_v.at[0]], csem).wait()
        pltpu.make_async_copy(ones_v, cnt_sh.at[dst_v.at[0]], csem).wait()
        return carry

    lax.fori_loop(0, (NCHUNK - 1) // 2, step, 0)
    pltpu.make_async_copy(y1_hbm.at[src_v.at[0]], rows0_v, gsem0).wait()
    pltpu.sync_copy(rows0_v, agg_sh.at[dst_v.at[NCHUNK - 1]], add=True)
    pltpu.sync_copy(ones_v, cnt_sh.at[dst_v.at[NCHUNK - 1]], add=True)
    plsc.subcore_barrier()

    pltpu.sync_copy(agg_sh.at[pl.ds(s * RP, RP)], stage_v)
    pltpu.sync_copy(cnt_sh.at[pl.ds(s * RP, RP)], cstage_v)
    pltpu.sync_copy(stage_v, p_hbm.at[c, pl.ds(s * RP, RP)])
    pltpu.sync_copy(cstage_v, cnt_hbm.at[c, pl.ds(s * RP, RP)])


def _mid_body(p_ref, cnt_ref, r1_ref, b1_ref, w2l_ref, w2r_ref,
              y2_ref, r2_ref):
    cnt = cnt_ref[0, :N] + cnt_ref[1, :N]
    rcp = 1.0 / jnp.maximum(cnt, 1.0)
    agg = p_ref[0, :N] + p_ref[1, :N]
    h = jax.nn.relu(agg * rcp[:, None] + r1_ref[...] + b1_ref[...][None, :])
    pad = jnp.zeros((NP - N,), jnp.float32)
    y2_ref[...] = jnp.concatenate(
        [jnp.sum(h * w2l_ref[...][0][None, :], axis=1), pad])
    r2_ref[...] = jnp.sum(h * w2r_ref[...][0][None, :], axis=1)


def _agg2_body(y2_hbm, src_hbm, dst_hbm, z1_hbm,
               q_hbm,
               q_sh, src_v, dst_v, vals_v, y2l_v, cstage_v,
               s0, s1, s2, s3):
    ssem = (s0, s1, s2, s3)
    c = lax.axis_index("c")
    s = lax.axis_index("s")
    wid = c * NS + s

    pltpu.sync_copy(z1_hbm, cstage_v)
    pltpu.sync_copy(cstage_v, q_sh.at[pl.ds(s * RP, RP)])
    pltpu.sync_copy(y2_hbm, y2l_v)
    pltpu.sync_copy(src_hbm.at[wid], src_v)
    pltpu.sync_copy(dst_hbm.at[wid], dst_v)
    plsc.subcore_barrier()

    def fill_and_scatter(ch, b):
        sv = src_v.at[ch]
        vb = vals_v.at[b]
        for j in range(CH // 16):
            idx = sv[pl.ds(j * 16, 16)]
            vb[pl.ds(j * 16, 16)] = plsc.load_gather(y2l_v, [idx])
        pltpu.async_copy(vb, q_sh.at[dst_v.at[ch]], ssem[b], add=True)

    for b in range(NB):
        fill_and_scatter(b, b)

    def round_(i, carry):
        for b in range(NB):
            ch = i * NB + b
            pltpu.make_async_copy(vals_v.at[b], q_sh.at[dst_v.at[0]],
                                  ssem[b]).wait()
            fill_and_scatter(ch, b)
        return carry

    lax.fori_loop(1, ROUNDS, round_, 0)
    for b in range(NB):
        pltpu.make_async_copy(vals_v.at[b], q_sh.at[dst_v.at[0]],
                              ssem[b]).wait()
    plsc.subcore_barrier()

    pltpu.sync_copy(q_sh.at[pl.ds(s * RP, RP)], cstage_v)
    pltpu.sync_copy(cstage_v, q_hbm.at[c, pl.ds(s * RP, RP)])


def _out_body(q_ref, cnt_ref, r2_ref, b2_ref, out_ref):
    cnt = cnt_ref[0, :N] + cnt_ref[1, :N]
    rcp = 1.0 / jnp.maximum(cnt, 1.0)
    z = (q_ref[0, :N] + q_ref[1, :N]) * rcp + r2_ref[...] + b2_ref[0]
    out_ref[...] = jax.nn.sigmoid(z)[:, None]


@jax.jit
def kernel(x, edge_index, W1l, W1r, b1, W2l, W2r, b2):
    f32 = jnp.float32
    i32 = jnp.int32
    src = edge_index[0].reshape(NW, NCHUNK, CH)
    dst = edge_index[1].reshape(NW, NCHUNK, CH)
    srcp = jnp.concatenate(
        [edge_index[0].reshape(NW, EPW), jnp.zeros((NW, PADW), i32)],
        axis=1).reshape(NW, NCHUNKP, CH)
    pad_dst = jnp.broadcast_to(jnp.arange(N, NP, dtype=i32)[None, :],
                               (NW, PADW))
    dstp = jnp.concatenate(
        [edge_index[1].reshape(NW, EPW), pad_dst],
        axis=1).reshape(NW, NCHUNKP, CH)
    z64 = jnp.zeros((RP, D_HID), f32)
    z1 = jnp.zeros((RP,), f32)
    ones = jnp.ones((CH,), f32)

    y1, r1 = pl.pallas_call(
        _proj1_body,
        out_shape=[jax.ShapeDtypeStruct((N, D_HID), f32),
                   jax.ShapeDtypeStruct((N, D_HID), f32)],
    )(x, W1l, W1r)

    agg1_partial, cnt_partial = pl.kernel(
        _agg1_body,
        out_type=[jax.ShapeDtypeStruct((NC, NP, D_HID), f32),
                  jax.ShapeDtypeStruct((NC, NP), f32)],
        mesh=_mesh,
        compiler_params=pltpu.CompilerParams(use_tc_tiling_on_sc=False),
        scratch_types=[
            pltpu.VMEM_SHARED((NP, D_HID), f32),
            pltpu.VMEM_SHARED((NP,), f32),
            pltpu.VMEM((NCHUNK, CH), i32),
            pltpu.VMEM((NCHUNK, CH), i32),
            pltpu.VMEM((CH, D_HID), f32),
            pltpu.VMEM((CH, D_HID), f32),
            pltpu.VMEM((CH,), f32),
            pltpu.VMEM((RP, D_HID), f32),
            pltpu.VMEM((RP,), f32),
        ] + [pltpu.SemaphoreType.DMA] * 5,
    )(y1, src, dst, z64, z1, ones)

    y2, r2 = pl.pallas_call(
        _mid_body,
        out_shape=[jax.ShapeDtypeStruct((NP,), f32),
                   jax.ShapeDtypeStruct((N,), f32)],
    )(agg1_partial, cnt_partial, r1, b1, W2l, W2r)

    q_partial = pl.kernel(
        _agg2_body,
        out_type=jax.ShapeDtypeStruct((NC, NP), f32),
        mesh=_mesh,
        compiler_params=pltpu.CompilerParams(use_tc_tiling_on_sc=False,
                                             needs_layout_passes=False),
        scratch_types=[
            pltpu.VMEM_SHARED((NP,), f32),
            pltpu.VMEM((NCHUNKP, CH), i32),
            pltpu.VMEM((NCHUNKP, CH), i32),
            pltpu.VMEM((NB, CH), f32),
            pltpu.VMEM((NP,), f32),
            pltpu.VMEM((RP,), f32),
        ] + [pltpu.SemaphoreType.DMA] * 4,
    )(y2, srcp, dstp, z1)

    out = pl.pallas_call(
        _out_body,
        out_shape=jax.ShapeDtypeStruct((N, 1), f32),
    )(q_partial, cnt_partial, r2, b2)

    return out
